# baseline (speedup 1.0000x reference)
import functools

import jax
import jax.numpy as jnp
from jax import lax
from jax.experimental import pallas as pl
from jax.experimental.pallas import tpu as pltpu
from jax.experimental.pallas import tpu_sc as plsc

_BLK = 256
_CHUNK = 16
_NWORK = 32
_FF_TILE = 2048


def _router_body(x_ref, gw_ref, gb_ref, idx_ref, cnt_ref, carry_ref,
                 *, n_experts):
    i = pl.program_id(0)

    @pl.when(i == 0)
    def _():
        carry_ref[...] = jnp.zeros_like(carry_ref)

    x = x_ref[...]
    logits = jnp.dot(x, gw_ref[...], preferred_element_type=jnp.float32)
    logits = logits + gb_ref[...]
    m = logits.shape[0]
    iota = lax.broadcasted_iota(jnp.int32, (m, n_experts), 1)
    m1 = jnp.max(logits, axis=1, keepdims=True)
    idx1 = jnp.min(jnp.where(logits == m1, iota, n_experts), axis=1,
                   keepdims=True)
    l2 = jnp.where(iota == idx1, -jnp.inf, logits)
    m2 = jnp.max(l2, axis=1, keepdims=True)
    idx2 = jnp.min(jnp.where(l2 == m2, iota, n_experts), axis=1, keepdims=True)
    mask = ((iota == idx1) | (iota == idx2)).astype(jnp.float32)

    r_iota = lax.broadcasted_iota(jnp.int32, (m, m), 0)
    c_iota = lax.broadcasted_iota(jnp.int32, (m, m), 1)
    tri = (r_iota > c_iota).astype(jnp.float32)
    excl = jnp.dot(tri, mask, preferred_element_type=jnp.float32)
    rank = excl + carry_ref[...]
    r0 = jnp.sum(jnp.where(iota == idx1, rank, 0.0), axis=1, keepdims=True)
    r1 = jnp.sum(jnp.where(iota == idx2, rank, 0.0), axis=1, keepdims=True)

    idx_ref[0, :, 0:1] = idx1
    idx_ref[0, :, 1:2] = idx2
    idx_ref[0, :, 2:3] = r0.astype(jnp.int32)
    idx_ref[0, :, 3:4] = r1.astype(jnp.int32)

    new_carry = carry_ref[...] + jnp.sum(mask, axis=0, keepdims=True)
    carry_ref[...] = new_carry
    cnt_ref[...] = new_carry.astype(jnp.int32)


def _route(xf, gate_W, gate_b):
    t, hid = xf.shape
    n_experts = gate_W.shape[1]
    grid = (t // _BLK,)
    return pl.pallas_call(
        functools.partial(_router_body, n_experts=n_experts),
        grid=grid,
        in_specs=[
            pl.BlockSpec((_BLK, hid), lambda i: (i, 0)),
            pl.BlockSpec((hid, n_experts), lambda i: (0, 0)),
            pl.BlockSpec((1, n_experts), lambda i: (0, 0)),
        ],
        out_specs=[
            pl.BlockSpec((1, _BLK, 4), lambda i: (i, 0, 0)),
            pl.BlockSpec((1, n_experts), lambda i: (0, 0)),
        ],
        out_shape=[
            jax.ShapeDtypeStruct((t // _BLK, _BLK, 4), jnp.int32),
            jax.ShapeDtypeStruct((1, n_experts), jnp.int32),
        ],
        scratch_shapes=[pltpu.VMEM((1, n_experts), jnp.float32)],
    )(xf, gate_W, gate_b.reshape(1, n_experts))


def _slots_body(idx_ref, st_ref, p_ref, *, n_experts):
    e0 = idx_ref[0, :, 0:1]
    e1 = idx_ref[0, :, 1:2]
    r0 = idx_ref[0, :, 2:3]
    r1 = idx_ref[0, :, 3:4]
    m = e0.shape[0]
    iota = lax.broadcasted_iota(jnp.int32, (m, n_experts), 1)
    st = st_ref[...]
    p_ref[0, :, 0:1] = r0 + jnp.sum(
        jnp.where(iota == e0, st, 0), axis=1, keepdims=True)
    p_ref[0, :, 1:2] = r1 + jnp.sum(
        jnp.where(iota == e1, st, 0), axis=1, keepdims=True)


def _slots(idx4, starts_rows):
    nblk_t, m, _ = idx4.shape
    n_experts = starts_rows.shape[1]
    return pl.pallas_call(
        functools.partial(_slots_body, n_experts=n_experts),
        grid=(nblk_t,),
        in_specs=[
            pl.BlockSpec((1, m, 4), lambda i: (i, 0, 0)),
            pl.BlockSpec((1, n_experts), lambda i: (0, 0)),
        ],
        out_specs=pl.BlockSpec((1, m, 2), lambda i: (i, 0, 0)),
        out_shape=jax.ShapeDtypeStruct((nblk_t, m, 2), jnp.int32),
    )(idx4, starts_rows)


def _sc_dispatch(xf, p0, p1, npad):
    t, hid = xf.shape
    tpw = t // _NWORK

    nchunk = tpw // _CHUNK

    @functools.partial(
        pl.kernel,
        mesh=plsc.VectorSubcoreMesh(core_axis_name="c", subcore_axis_name="s"),
        out_type=jax.ShapeDtypeStruct((npad, hid), jnp.float32),
        scratch_types=[
            pltpu.VMEM((tpw,), jnp.int32),
            pltpu.VMEM((tpw,), jnp.int32),
            pltpu.VMEM((4, _CHUNK, hid), jnp.float32),
            pltpu.SemaphoreType.DMA,
            pltpu.SemaphoreType.DMA,
            pltpu.SemaphoreType.DMA,
            pltpu.SemaphoreType.DMA,
            pltpu.SemaphoreType.DMA,
            pltpu.SemaphoreType.DMA,
            pltpu.SemaphoreType.DMA,
            pltpu.SemaphoreType.DMA,
        ],
    )
    def body(x_hbm, p0_hbm, p1_hbm, xs_hbm, p0v, p1v, xb,
             si0, si1, si2, si3, ss0, ss1, ss2, ss3):
        wid = lax.axis_index("s") * 2 + lax.axis_index("c")
        base = wid * tpw
        pltpu.sync_copy(p0_hbm.at[pl.ds(base, tpw)], p0v)
        pltpu.sync_copy(p1_hbm.at[pl.ds(base, tpw)], p1v)
        depth = 4
        si = (si0, si1, si2, si3)
        ss = (ss0, ss1, ss2, ss3)

        def load(c):
            return pltpu.async_copy(
                x_hbm.at[pl.ds(base + c * _CHUNK, _CHUNK)],
                xb.at[c % depth], si[c % depth])

        in_h = {c: load(c) for c in range(min(depth - 1, nchunk))}
        sc_h = {}
        for c in range(nchunk):
            b = c % depth
            sl = pl.ds(c * _CHUNK, _CHUNK)
            in_h.pop(c).wait()
            sc_h[c] = (
                pltpu.async_copy(xb.at[b], xs_hbm.at[p0v[sl]], ss[b]),
                pltpu.async_copy(xb.at[b], xs_hbm.at[p1v[sl]], ss[b]))
            nxt = c + depth - 1
            if nxt < nchunk:
                if c >= 1:
                    h = sc_h.pop(c - 1)
                    h[0].wait()
                    h[1].wait()
                in_h[nxt] = load(nxt)
        for c in sorted(sc_h):
            h = sc_h.pop(c)
            h[0].wait()
            h[1].wait()

    return body(xf, p0, p1)


def _gemm_body(g_ref, act_ref, xs_ref, w1_ref, b1_ref, w2_ref, b2_ref, ys_ref):
    i = pl.program_id(0)

    @pl.when(act_ref[i] == 1)
    def _():
        xb = xs_ref[...].astype(jnp.bfloat16)
        ff = w1_ref.shape[2]
        acc = jnp.zeros(ys_ref.shape, jnp.float32)
        for j in range(ff // _FF_TILE):
            sl = slice(j * _FF_TILE, (j + 1) * _FF_TILE)
            h = jnp.dot(xb, w1_ref[0, :, sl].astype(jnp.bfloat16),
                        preferred_element_type=jnp.float32)
            h = jnp.maximum(h + b1_ref[0, 0, sl][None, :], 0.0)
            acc = acc + jnp.dot(h.astype(jnp.bfloat16), w2_ref[0, sl, :],
                                preferred_element_type=jnp.float32)
        ys_ref[...] = acc + b2_ref[0, 0, :][None, :]


def _grouped_gemm(g2, act, xs, w1b, b1r, w2b, b2r, nblk):
    npad, hid = xs.shape
    n_experts, _, ff = w1b.shape
    return pl.pallas_call(
        _gemm_body,
        grid_spec=pltpu.PrefetchScalarGridSpec(
            num_scalar_prefetch=2,
            grid=(nblk,),
            in_specs=[
                pl.BlockSpec((_BLK, hid), lambda i, g, a: (i, 0)),
                pl.BlockSpec((1, hid, ff), lambda i, g, a: (g[i], 0, 0)),
                pl.BlockSpec((1, 1, ff), lambda i, g, a: (g[i], 0, 0)),
                pl.BlockSpec((1, ff, hid), lambda i, g, a: (g[i], 0, 0)),
                pl.BlockSpec((1, 1, hid), lambda i, g, a: (g[i], 0, 0)),
            ],
            out_specs=pl.BlockSpec((_BLK, hid), lambda i, g, a: (i, 0)),
        ),
        out_shape=jax.ShapeDtypeStruct((npad, hid), jnp.float32),
        compiler_params=pltpu.CompilerParams(
            vmem_limit_bytes=63 * 1024 * 1024),
    )(g2, act, xs, w1b, b1r, w2b, b2r)


def _sc_combine(ys, p0, p1, t, hid):
    tpw = t // _NWORK

    nchunk = tpw // _CHUNK

    @functools.partial(
        pl.kernel,
        mesh=plsc.VectorSubcoreMesh(core_axis_name="c", subcore_axis_name="s"),
        out_type=jax.ShapeDtypeStruct((t, hid), jnp.float32),
        scratch_types=[
            pltpu.VMEM((tpw,), jnp.int32),
            pltpu.VMEM((tpw,), jnp.int32),
            pltpu.VMEM((3, _CHUNK, hid), jnp.float32),
            pltpu.VMEM((3, _CHUNK, hid), jnp.float32),
            pltpu.SemaphoreType.DMA,
            pltpu.SemaphoreType.DMA,
            pltpu.SemaphoreType.DMA,
            pltpu.SemaphoreType.DMA,
            pltpu.SemaphoreType.DMA,
            pltpu.SemaphoreType.DMA,
        ],
    )
    def body(ys_hbm, p0_hbm, p1_hbm, out_hbm, p0v, p1v, av, bv,
             sg0, sg1, sg2, so0, so1, so2):
        wid = lax.axis_index("s") * 2 + lax.axis_index("c")
        base = wid * tpw
        pltpu.sync_copy(p0_hbm.at[pl.ds(base, tpw)], p0v)
        pltpu.sync_copy(p1_hbm.at[pl.ds(base, tpw)], p1v)
        depth = 3
        sg = (sg0, sg1, sg2)
        so = (so0, so1, so2)

        def gathers(c):
            b = c % depth
            sl = pl.ds(c * _CHUNK, _CHUNK)
            return (pltpu.async_copy(ys_hbm.at[p0v[sl]], av.at[b], sg[b]),
                    pltpu.async_copy(ys_hbm.at[p1v[sl]], bv.at[b], sg[b]))

        g_h = {c: gathers(c) for c in range(min(depth - 1, nchunk))}
        out_h = {}
        for c in range(nchunk):
            b = c % depth
            h0, h1 = g_h.pop(c)
            h0.wait()
            h1.wait()
            nxt = c + depth - 1
            if nxt < nchunk:
                if c >= 1:
                    out_h.pop(c - 1).wait()
                g_h[nxt] = gathers(nxt)

            def add_lane(k, _):
                dsl = pl.ds(k * _CHUNK, _CHUNK)
                for r in range(_CHUNK):
                    av[b, r, dsl] = av[b, r, dsl] + bv[b, r, dsl]
                return 0

            lax.fori_loop(0, hid // _CHUNK, add_lane, 0)
            out_h[c] = pltpu.async_copy(
                av.at[b], out_hbm.at[pl.ds(base + c * _CHUNK, _CHUNK)], so[b])
        for c in sorted(out_h):
            out_h.pop(c).wait()

    return body(ys, p0, p1)


def kernel(x, gate_W, gate_b, W1, b1, W2, b2):
    bsz, seq, hid = x.shape
    t = bsz * seq
    n_experts = gate_W.shape[1]
    ff = W1.shape[2]
    topk = 2
    xf = x.reshape(t, hid)
    nblk = t * topk // _BLK + n_experts
    npad = nblk * _BLK

    idx4, counts = _route(xf, gate_W, gate_b)
    counts = counts.reshape(n_experts)

    nb = (counts + _BLK - 1) // _BLK
    bstart = jnp.concatenate(
        [jnp.zeros((1,), jnp.int32), jnp.cumsum(nb)[:-1].astype(jnp.int32)])
    nba = jnp.sum(nb).astype(jnp.int32)
    ii = jnp.arange(nblk, dtype=jnp.int32)
    iic = jnp.minimum(ii, nba - 1)
    g2 = jnp.sum((bstart[None, :] <= iic[:, None]).astype(jnp.int32),
                 axis=1) - 1
    act = (ii < nba).astype(jnp.int32)
    starts_rows = (bstart * _BLK).astype(jnp.int32)

    p01 = _slots(idx4, starts_rows.reshape(1, n_experts))
    p0 = p01[:, :, 0].reshape(t)
    p1 = p01[:, :, 1].reshape(t)

    xs = _sc_dispatch(xf, p0, p1, npad)

    ys = _grouped_gemm(g2, act, xs, W1, b1.reshape(n_experts, 1, ff),
                       W2.astype(jnp.bfloat16), b2.reshape(n_experts, 1, hid),
                       nblk)

    out = _sc_combine(ys, p0, p1, t, hid)
    return out.reshape(bsz, seq, hid)


# speedup vs baseline: 1.0048x; 1.0048x over previous
import functools

import jax
import jax.numpy as jnp
from jax import lax
from jax.experimental import pallas as pl
from jax.experimental.pallas import tpu as pltpu
from jax.experimental.pallas import tpu_sc as plsc

_BLK = 256
_CHUNK = 16
_NWORK = 32
_FF_TILE = 2048


def _router_body(x_ref, gw_ref, gb_ref, idx_ref, cnt_ref, carry_ref,
                 *, n_experts):
    i = pl.program_id(0)

    @pl.when(i == 0)
    def _():
        carry_ref[...] = jnp.zeros_like(carry_ref)

    x = x_ref[...]
    logits = jnp.dot(x, gw_ref[...], preferred_element_type=jnp.float32)
    logits = logits + gb_ref[...]
    m = logits.shape[0]
    iota = lax.broadcasted_iota(jnp.int32, (m, n_experts), 1)
    m1 = jnp.max(logits, axis=1, keepdims=True)
    idx1 = jnp.min(jnp.where(logits == m1, iota, n_experts), axis=1,
                   keepdims=True)
    l2 = jnp.where(iota == idx1, -jnp.inf, logits)
    m2 = jnp.max(l2, axis=1, keepdims=True)
    idx2 = jnp.min(jnp.where(l2 == m2, iota, n_experts), axis=1, keepdims=True)
    mask = ((iota == idx1) | (iota == idx2)).astype(jnp.float32)

    r_iota = lax.broadcasted_iota(jnp.int32, (m, m), 0)
    c_iota = lax.broadcasted_iota(jnp.int32, (m, m), 1)
    tri = (r_iota > c_iota).astype(jnp.float32)
    excl = jnp.dot(tri, mask, preferred_element_type=jnp.float32)
    rank = excl + carry_ref[...]
    r0 = jnp.sum(jnp.where(iota == idx1, rank, 0.0), axis=1, keepdims=True)
    r1 = jnp.sum(jnp.where(iota == idx2, rank, 0.0), axis=1, keepdims=True)

    idx_ref[0, :, 0:1] = idx1
    idx_ref[0, :, 1:2] = idx2
    idx_ref[0, :, 2:3] = r0.astype(jnp.int32)
    idx_ref[0, :, 3:4] = r1.astype(jnp.int32)

    new_carry = carry_ref[...] + jnp.sum(mask, axis=0, keepdims=True)
    carry_ref[...] = new_carry
    cnt_ref[...] = new_carry.astype(jnp.int32)


def _route(xf, gate_W, gate_b):
    t, hid = xf.shape
    n_experts = gate_W.shape[1]
    grid = (t // _BLK,)
    return pl.pallas_call(
        functools.partial(_router_body, n_experts=n_experts),
        grid=grid,
        in_specs=[
            pl.BlockSpec((_BLK, hid), lambda i: (i, 0)),
            pl.BlockSpec((hid, n_experts), lambda i: (0, 0)),
            pl.BlockSpec((1, n_experts), lambda i: (0, 0)),
        ],
        out_specs=[
            pl.BlockSpec((1, _BLK, 4), lambda i: (i, 0, 0)),
            pl.BlockSpec((1, n_experts), lambda i: (0, 0)),
        ],
        out_shape=[
            jax.ShapeDtypeStruct((t // _BLK, _BLK, 4), jnp.int32),
            jax.ShapeDtypeStruct((1, n_experts), jnp.int32),
        ],
        scratch_shapes=[pltpu.VMEM((1, n_experts), jnp.float32)],
    )(xf, gate_W, gate_b.reshape(1, n_experts))


def _slots_body(idx_ref, st_ref, p_ref, *, n_experts):
    e0 = idx_ref[0, :, 0:1]
    e1 = idx_ref[0, :, 1:2]
    r0 = idx_ref[0, :, 2:3]
    r1 = idx_ref[0, :, 3:4]
    m = e0.shape[0]
    iota = lax.broadcasted_iota(jnp.int32, (m, n_experts), 1)
    st = st_ref[...]
    p_ref[0, :, 0:1] = r0 + jnp.sum(
        jnp.where(iota == e0, st, 0), axis=1, keepdims=True)
    p_ref[0, :, 1:2] = r1 + jnp.sum(
        jnp.where(iota == e1, st, 0), axis=1, keepdims=True)


def _slots(idx4, starts_rows):
    nblk_t, m, _ = idx4.shape
    n_experts = starts_rows.shape[1]
    return pl.pallas_call(
        functools.partial(_slots_body, n_experts=n_experts),
        grid=(nblk_t,),
        in_specs=[
            pl.BlockSpec((1, m, 4), lambda i: (i, 0, 0)),
            pl.BlockSpec((1, n_experts), lambda i: (0, 0)),
        ],
        out_specs=pl.BlockSpec((1, m, 2), lambda i: (i, 0, 0)),
        out_shape=jax.ShapeDtypeStruct((nblk_t, m, 2), jnp.int32),
    )(idx4, starts_rows)


def _sc_dispatch(xf, p0, p1, npad):
    t, hid = xf.shape
    tpw = t // _NWORK

    nchunk = tpw // _CHUNK

    @functools.partial(
        pl.kernel,
        mesh=plsc.VectorSubcoreMesh(core_axis_name="c", subcore_axis_name="s"),
        out_type=jax.ShapeDtypeStruct((npad, hid), jnp.float32),
        scratch_types=[
            pltpu.VMEM((tpw,), jnp.int32),
            pltpu.VMEM((tpw,), jnp.int32),
            pltpu.VMEM((2, _CHUNK, hid), jnp.float32),
            pltpu.SemaphoreType.DMA,
            pltpu.SemaphoreType.DMA,
            pltpu.SemaphoreType.DMA,
            pltpu.SemaphoreType.DMA,
        ],
    )
    def body(x_hbm, p0_hbm, p1_hbm, xs_hbm, p0v, p1v, xb, si0, si1, ss0, ss1):
        wid = lax.axis_index("s") * 2 + lax.axis_index("c")
        base = wid * tpw
        pltpu.sync_copy(p0_hbm.at[pl.ds(base, tpw)], p0v)
        pltpu.sync_copy(p1_hbm.at[pl.ds(base, tpw)], p1v)
        si = (si0, si1)
        ss = (ss0, ss1)

        def load(c):
            return pltpu.async_copy(
                x_hbm.at[pl.ds(base + c * _CHUNK, _CHUNK)],
                xb.at[c % 2], si[c % 2])

        in_h = {0: load(0)}
        prev = None
        for c in range(nchunk):
            b = c % 2
            sl = pl.ds(c * _CHUNK, _CHUNK)
            in_h.pop(c).wait()
            s0 = pltpu.async_copy(xb.at[b], xs_hbm.at[p0v[sl]], ss[b])
            s1 = pltpu.async_copy(xb.at[b], xs_hbm.at[p1v[sl]], ss[b])
            if prev is not None:
                prev[0].wait()
                prev[1].wait()
            if c + 1 < nchunk:
                in_h[c + 1] = load(c + 1)
            prev = (s0, s1)
        prev[0].wait()
        prev[1].wait()

    return body(xf, p0, p1)


def _gemm_body(g_ref, act_ref, xs_ref, w1_ref, b1_ref, w2_ref, b2_ref, ys_ref):
    i = pl.program_id(0)

    @pl.when(act_ref[i] == 1)
    def _():
        xb = xs_ref[...].astype(jnp.bfloat16)
        ff = w1_ref.shape[2]
        acc = jnp.zeros(ys_ref.shape, jnp.float32)
        for j in range(ff // _FF_TILE):
            sl = slice(j * _FF_TILE, (j + 1) * _FF_TILE)
            h = jnp.dot(xb, w1_ref[0, :, sl].astype(jnp.bfloat16),
                        preferred_element_type=jnp.float32)
            h = jnp.maximum(h + b1_ref[0, 0, sl][None, :], 0.0)
            acc = acc + jnp.dot(h.astype(jnp.bfloat16), w2_ref[0, sl, :],
                                preferred_element_type=jnp.float32)
        ys_ref[...] = acc + b2_ref[0, 0, :][None, :]


def _grouped_gemm(g2, act, xs, w1b, b1r, w2b, b2r, nblk):
    npad, hid = xs.shape
    n_experts, _, ff = w1b.shape
    return pl.pallas_call(
        _gemm_body,
        grid_spec=pltpu.PrefetchScalarGridSpec(
            num_scalar_prefetch=2,
            grid=(nblk,),
            in_specs=[
                pl.BlockSpec((_BLK, hid), lambda i, g, a: (i, 0)),
                pl.BlockSpec((1, hid, ff), lambda i, g, a: (g[i], 0, 0)),
                pl.BlockSpec((1, 1, ff), lambda i, g, a: (g[i], 0, 0)),
                pl.BlockSpec((1, ff, hid), lambda i, g, a: (g[i], 0, 0)),
                pl.BlockSpec((1, 1, hid), lambda i, g, a: (g[i], 0, 0)),
            ],
            out_specs=pl.BlockSpec((_BLK, hid), lambda i, g, a: (i, 0)),
        ),
        out_shape=jax.ShapeDtypeStruct((npad, hid), jnp.float32),
        compiler_params=pltpu.CompilerParams(
            vmem_limit_bytes=63 * 1024 * 1024),
    )(g2, act, xs, w1b, b1r, w2b, b2r)


def _sc_combine(ys, p0, p1, t, hid):
    tpw = t // _NWORK

    nchunk = tpw // _CHUNK

    @functools.partial(
        pl.kernel,
        mesh=plsc.VectorSubcoreMesh(core_axis_name="c", subcore_axis_name="s"),
        out_type=jax.ShapeDtypeStruct((t, hid), jnp.float32),
        scratch_types=[
            pltpu.VMEM((tpw,), jnp.int32),
            pltpu.VMEM((tpw,), jnp.int32),
            pltpu.VMEM((2, _CHUNK, hid), jnp.float32),
            pltpu.VMEM((2, _CHUNK, hid), jnp.float32),
            pltpu.SemaphoreType.DMA,
            pltpu.SemaphoreType.DMA,
            pltpu.SemaphoreType.DMA,
            pltpu.SemaphoreType.DMA,
        ],
    )
    def body(ys_hbm, p0_hbm, p1_hbm, out_hbm, p0v, p1v, av, bv,
             sg0, sg1, so0, so1):
        wid = lax.axis_index("s") * 2 + lax.axis_index("c")
        base = wid * tpw
        pltpu.sync_copy(p0_hbm.at[pl.ds(base, tpw)], p0v)
        pltpu.sync_copy(p1_hbm.at[pl.ds(base, tpw)], p1v)
        sg = (sg0, sg1)
        so = (so0, so1)

        def gathers(c):
            b = c % 2
            sl = pl.ds(c * _CHUNK, _CHUNK)
            return (pltpu.async_copy(ys_hbm.at[p0v[sl]], av.at[b], sg[b]),
                    pltpu.async_copy(ys_hbm.at[p1v[sl]], bv.at[b], sg[b]))

        g_h = {0: gathers(0)}
        out_h = {}
        for c in range(nchunk):
            b = c % 2
            h0, h1 = g_h.pop(c)
            h0.wait()
            h1.wait()
            if c + 1 < nchunk:
                if c >= 1:
                    out_h.pop(c - 1).wait()
                g_h[c + 1] = gathers(c + 1)

            def add_lane(k, _):
                dsl = pl.ds(k * _CHUNK, _CHUNK)
                for r in range(_CHUNK):
                    av[b, r, dsl] = av[b, r, dsl] + bv[b, r, dsl]
                return 0

            lax.fori_loop(0, hid // _CHUNK, add_lane, 0)
            out_h[c] = pltpu.async_copy(
                av.at[b], out_hbm.at[pl.ds(base + c * _CHUNK, _CHUNK)], so[b])
        for c in sorted(out_h):
            out_h.pop(c).wait()

    return body(ys, p0, p1)


def kernel(x, gate_W, gate_b, W1, b1, W2, b2):
    bsz, seq, hid = x.shape
    t = bsz * seq
    n_experts = gate_W.shape[1]
    ff = W1.shape[2]
    topk = 2
    xf = x.reshape(t, hid)
    nblk = t * topk // _BLK + n_experts
    npad = nblk * _BLK

    idx4, counts = _route(xf, gate_W, gate_b)
    counts = counts.reshape(n_experts)

    nb = (counts + _BLK - 1) // _BLK
    bstart = jnp.concatenate(
        [jnp.zeros((1,), jnp.int32), jnp.cumsum(nb)[:-1].astype(jnp.int32)])
    nba = jnp.sum(nb).astype(jnp.int32)
    ii = jnp.arange(nblk, dtype=jnp.int32)
    iic = jnp.minimum(ii, nba - 1)
    g2 = jnp.sum((bstart[None, :] <= iic[:, None]).astype(jnp.int32),
                 axis=1) - 1
    act = (ii < nba).astype(jnp.int32)
    starts_rows = (bstart * _BLK).astype(jnp.int32)

    p01 = _slots(idx4, starts_rows.reshape(1, n_experts))
    p0 = p01[:, :, 0].reshape(t)
    p1 = p01[:, :, 1].reshape(t)

    xs = _sc_dispatch(xf, p0, p1, npad)

    ys = _grouped_gemm(g2, act, xs, W1, b1.reshape(n_experts, 1, ff),
                       W2.astype(jnp.bfloat16), b2.reshape(n_experts, 1, hid),
                       nblk)

    out = _sc_combine(ys, p0, p1, t, hid)
    return out.reshape(bsz, seq, hid)


# speedup vs baseline: 1.0311x; 1.0261x over previous
import functools

import jax
import jax.numpy as jnp
from jax import lax
from jax.experimental import pallas as pl
from jax.experimental.pallas import tpu as pltpu
from jax.experimental.pallas import tpu_sc as plsc

_BLK = 256
_CHUNK = 16
_NWORK = 32
_FF_TILE = 2048


def _router_body(x_ref, gw_ref, gb_ref, p_ref, cnt_ref, carry_ref, idx_ref,
                 *, n_experts, blk):
    i = pl.program_id(0)

    @pl.when(i == 0)
    def _():
        carry_ref[...] = jnp.zeros_like(carry_ref)

    x = x_ref[...]
    logits = jnp.dot(x, gw_ref[...], preferred_element_type=jnp.float32)
    logits = logits + gb_ref[...]
    m = logits.shape[0]
    iota = lax.broadcasted_iota(jnp.int32, (m, n_experts), 1)
    m1 = jnp.max(logits, axis=1, keepdims=True)
    idx1 = jnp.min(jnp.where(logits == m1, iota, n_experts), axis=1,
                   keepdims=True)
    l2 = jnp.where(iota == idx1, -jnp.inf, logits)
    m2 = jnp.max(l2, axis=1, keepdims=True)
    idx2 = jnp.min(jnp.where(l2 == m2, iota, n_experts), axis=1, keepdims=True)
    mask = ((iota == idx1) | (iota == idx2)).astype(jnp.float32)

    r_iota = lax.broadcasted_iota(jnp.int32, (m, m), 0)
    c_iota = lax.broadcasted_iota(jnp.int32, (m, m), 1)
    tri = (r_iota > c_iota).astype(jnp.float32)
    excl = jnp.dot(tri, mask, preferred_element_type=jnp.float32)
    rank = excl + carry_ref[...]
    r0 = jnp.sum(jnp.where(iota == idx1, rank, 0.0), axis=1, keepdims=True)
    r1 = jnp.sum(jnp.where(iota == idx2, rank, 0.0), axis=1, keepdims=True)

    idx_ref[i, :, 0:1] = idx1
    idx_ref[i, :, 1:2] = idx2
    idx_ref[i, :, 2:3] = r0.astype(jnp.int32)
    idx_ref[i, :, 3:4] = r1.astype(jnp.int32)

    new_carry = carry_ref[...] + jnp.sum(mask, axis=0, keepdims=True)
    carry_ref[...] = new_carry
    cnt_ref[...] = new_carry.astype(jnp.int32)

    nblk_t = pl.num_programs(0)

    @pl.when(i == nblk_t - 1)
    def _():
        cnt = new_carry
        nb = jnp.floor((cnt + (blk - 1)) / blk)
        r8 = lax.broadcasted_iota(jnp.int32, (n_experts, n_experts), 0)
        c8 = lax.broadcasted_iota(jnp.int32, (n_experts, n_experts), 1)
        tri8 = (r8 < c8).astype(jnp.float32)
        starts = jnp.dot(nb, tri8, preferred_element_type=jnp.float32) * blk
        t_all = nblk_t * blk
        allv = idx_ref[...].reshape(t_all, 4)
        iota_e = lax.broadcasted_iota(jnp.int32, (t_all, n_experts), 1)
        st_i = starts.astype(jnp.int32)
        p0 = allv[:, 2:3] + jnp.sum(
            jnp.where(iota_e == allv[:, 0:1], st_i, 0), axis=1, keepdims=True)
        p1 = allv[:, 3:4] + jnp.sum(
            jnp.where(iota_e == allv[:, 1:2], st_i, 0), axis=1, keepdims=True)
        p_ref[...] = jnp.concatenate([p0, p1], axis=1).reshape(
            nblk_t, blk, 2)


def _route(xf, gate_W, gate_b):
    t, hid = xf.shape
    n_experts = gate_W.shape[1]
    grid = (t // _BLK,)
    nblk_t = t // _BLK
    return pl.pallas_call(
        functools.partial(_router_body, n_experts=n_experts, blk=_BLK),
        grid=grid,
        in_specs=[
            pl.BlockSpec((_BLK, hid), lambda i: (i, 0)),
            pl.BlockSpec((hid, n_experts), lambda i: (0, 0)),
            pl.BlockSpec((1, n_experts), lambda i: (0, 0)),
        ],
        out_specs=[
            pl.BlockSpec((nblk_t, _BLK, 2), lambda i: (0, 0, 0)),
            pl.BlockSpec((1, n_experts), lambda i: (0, 0)),
        ],
        out_shape=[
            jax.ShapeDtypeStruct((nblk_t, _BLK, 2), jnp.int32),
            jax.ShapeDtypeStruct((1, n_experts), jnp.int32),
        ],
        scratch_shapes=[pltpu.VMEM((1, n_experts), jnp.float32),
                        pltpu.VMEM((nblk_t, _BLK, 4), jnp.int32)],
    )(xf, gate_W, gate_b.reshape(1, n_experts))


def _sc_dispatch(xf, p0, p1, npad):
    t, hid = xf.shape
    tpw = t // _NWORK

    nchunk = tpw // _CHUNK

    @functools.partial(
        pl.kernel,
        mesh=plsc.VectorSubcoreMesh(core_axis_name="c", subcore_axis_name="s"),
        out_type=jax.ShapeDtypeStruct((npad, hid), jnp.float32),
        scratch_types=[
            pltpu.VMEM((tpw,), jnp.int32),
            pltpu.VMEM((tpw,), jnp.int32),
            pltpu.VMEM((2, _CHUNK, hid), jnp.float32),
            pltpu.SemaphoreType.DMA,
            pltpu.SemaphoreType.DMA,
            pltpu.SemaphoreType.DMA,
            pltpu.SemaphoreType.DMA,
        ],
    )
    def body(x_hbm, p0_hbm, p1_hbm, xs_hbm, p0v, p1v, xb, si0, si1, ss0, ss1):
        wid = lax.axis_index("s") * 2 + lax.axis_index("c")
        base = wid * tpw
        pltpu.sync_copy(p0_hbm.at[pl.ds(base, tpw)], p0v)
        pltpu.sync_copy(p1_hbm.at[pl.ds(base, tpw)], p1v)
        si = (si0, si1)
        ss = (ss0, ss1)

        def load(c):
            return pltpu.async_copy(
                x_hbm.at[pl.ds(base + c * _CHUNK, _CHUNK)],
                xb.at[c % 2], si[c % 2])

        in_h = {0: load(0)}
        prev = None
        for c in range(nchunk):
            b = c % 2
            sl = pl.ds(c * _CHUNK, _CHUNK)
            in_h.pop(c).wait()
            s0 = pltpu.async_copy(xb.at[b], xs_hbm.at[p0v[sl]], ss[b])
            s1 = pltpu.async_copy(xb.at[b], xs_hbm.at[p1v[sl]], ss[b])
            if prev is not None:
                prev[0].wait()
                prev[1].wait()
            if c + 1 < nchunk:
                in_h[c + 1] = load(c + 1)
            prev = (s0, s1)
        prev[0].wait()
        prev[1].wait()

    return body(xf, p0, p1)


def _gemm_body(g_ref, act_ref, xs_ref, w1_ref, b1_ref, w2_ref, b2_ref, ys_ref):
    i = pl.program_id(0)

    @pl.when(act_ref[i] == 1)
    def _():
        xb = xs_ref[...].astype(jnp.bfloat16)
        ff = w1_ref.shape[2]
        acc = jnp.zeros(ys_ref.shape, jnp.float32)
        for j in range(ff // _FF_TILE):
            sl = slice(j * _FF_TILE, (j + 1) * _FF_TILE)
            h = jnp.dot(xb, w1_ref[0, :, sl].astype(jnp.bfloat16),
                        preferred_element_type=jnp.float32)
            h = jnp.maximum(h + b1_ref[0, 0, sl][None, :], 0.0)
            acc = acc + jnp.dot(h.astype(jnp.bfloat16), w2_ref[0, sl, :],
                                preferred_element_type=jnp.float32)
        ys_ref[...] = acc + b2_ref[0, 0, :][None, :]


def _grouped_gemm(g2, act, xs, w1b, b1r, w2b, b2r, nblk):
    npad, hid = xs.shape
    n_experts, _, ff = w1b.shape
    return pl.pallas_call(
        _gemm_body,
        grid_spec=pltpu.PrefetchScalarGridSpec(
            num_scalar_prefetch=2,
            grid=(nblk,),
            in_specs=[
                pl.BlockSpec((_BLK, hid), lambda i, g, a: (i, 0)),
                pl.BlockSpec((1, hid, ff), lambda i, g, a: (g[i], 0, 0)),
                pl.BlockSpec((1, 1, ff), lambda i, g, a: (g[i], 0, 0)),
                pl.BlockSpec((1, ff, hid), lambda i, g, a: (g[i], 0, 0)),
                pl.BlockSpec((1, 1, hid), lambda i, g, a: (g[i], 0, 0)),
            ],
            out_specs=pl.BlockSpec((_BLK, hid), lambda i, g, a: (i, 0)),
        ),
        out_shape=jax.ShapeDtypeStruct((npad, hid), jnp.float32),
        compiler_params=pltpu.CompilerParams(
            vmem_limit_bytes=63 * 1024 * 1024),
    )(g2, act, xs, w1b, b1r, w2b, b2r)


def _sc_combine(ys, p0, p1, t, hid):
    tpw = t // _NWORK

    nchunk = tpw // _CHUNK

    @functools.partial(
        pl.kernel,
        mesh=plsc.VectorSubcoreMesh(core_axis_name="c", subcore_axis_name="s"),
        out_type=jax.ShapeDtypeStruct((t, hid), jnp.float32),
        scratch_types=[
            pltpu.VMEM((tpw,), jnp.int32),
            pltpu.VMEM((tpw,), jnp.int32),
            pltpu.VMEM((2, _CHUNK, hid), jnp.float32),
            pltpu.VMEM((2, _CHUNK, hid), jnp.float32),
            pltpu.SemaphoreType.DMA,
            pltpu.SemaphoreType.DMA,
            pltpu.SemaphoreType.DMA,
            pltpu.SemaphoreType.DMA,
        ],
    )
    def body(ys_hbm, p0_hbm, p1_hbm, out_hbm, p0v, p1v, av, bv,
             sg0, sg1, so0, so1):
        wid = lax.axis_index("s") * 2 + lax.axis_index("c")
        base = wid * tpw
        pltpu.sync_copy(p0_hbm.at[pl.ds(base, tpw)], p0v)
        pltpu.sync_copy(p1_hbm.at[pl.ds(base, tpw)], p1v)
        sg = (sg0, sg1)
        so = (so0, so1)

        def gathers(c):
            b = c % 2
            sl = pl.ds(c * _CHUNK, _CHUNK)
            return (pltpu.async_copy(ys_hbm.at[p0v[sl]], av.at[b], sg[b]),
                    pltpu.async_copy(ys_hbm.at[p1v[sl]], bv.at[b], sg[b]))

        g_h = {0: gathers(0)}
        out_h = {}
        for c in range(nchunk):
            b = c % 2
            h0, h1 = g_h.pop(c)
            h0.wait()
            h1.wait()
            if c + 1 < nchunk:
                if c >= 1:
                    out_h.pop(c - 1).wait()
                g_h[c + 1] = gathers(c + 1)

            def add_lane(k, _):
                dsl = pl.ds(k * _CHUNK, _CHUNK)
                for r in range(_CHUNK):
                    av[b, r, dsl] = av[b, r, dsl] + bv[b, r, dsl]
                return 0

            lax.fori_loop(0, hid // _CHUNK, add_lane, 0)
            out_h[c] = pltpu.async_copy(
                av.at[b], out_hbm.at[pl.ds(base + c * _CHUNK, _CHUNK)], so[b])
        for c in sorted(out_h):
            out_h.pop(c).wait()

    return body(ys, p0, p1)


def kernel(x, gate_W, gate_b, W1, b1, W2, b2):
    bsz, seq, hid = x.shape
    t = bsz * seq
    n_experts = gate_W.shape[1]
    ff = W1.shape[2]
    topk = 2
    xf = x.reshape(t, hid)
    nblk = t * topk // _BLK + n_experts
    npad = nblk * _BLK

    p01, counts = _route(xf, gate_W, gate_b)
    counts = counts.reshape(n_experts)

    nb = (counts + _BLK - 1) // _BLK
    bstart = jnp.concatenate(
        [jnp.zeros((1,), jnp.int32), jnp.cumsum(nb)[:-1].astype(jnp.int32)])
    nba = jnp.sum(nb).astype(jnp.int32)
    ii = jnp.arange(nblk, dtype=jnp.int32)
    iic = jnp.minimum(ii, nba - 1)
    g2 = jnp.sum((bstart[None, :] <= iic[:, None]).astype(jnp.int32),
                 axis=1) - 1
    act = (ii < nba).astype(jnp.int32)

    p0 = p01[:, :, 0].reshape(t)
    p1 = p01[:, :, 1].reshape(t)

    xs = _sc_dispatch(xf, p0, p1, npad)

    ys = _grouped_gemm(g2, act, xs, W1, b1.reshape(n_experts, 1, ff),
                       W2.astype(jnp.bfloat16), b2.reshape(n_experts, 1, hid),
                       nblk)

    out = _sc_combine(ys, p0, p1, t, hid)
    return out.reshape(bsz, seq, hid)
